# Initial kernel scaffold; baseline (speedup 1.0000x reference)
import functools

import jax
import jax.numpy as jnp
from jax import lax
from jax.experimental import pallas as pl
from jax.experimental.pallas import tpu as pltpu
from jax.experimental.pallas import tpu_sc as plsc

_D = 64
_NW = 32
_CHUNK = 128


def _body(nch, idx_hbm, expr_hbm, table_hbm, out_hbm, idx_v, expr_v, rows_v,
          sem):
  nc = plsc.get_sparse_core_info().num_cores
  wid = lax.axis_index("s") * nc + lax.axis_index("c")
  per_w = nch * _CHUNK
  base = wid * per_w

  pltpu.sync_copy(idx_hbm.at[pl.ds(wid * nch, nch)], idx_v)
  pltpu.sync_copy(expr_hbm.at[pl.ds(wid * nch, nch)], expr_v)

  def chunk_body(j):
    pltpu.async_copy(table_hbm.at[idx_v.at[j]], rows_v, sem).wait()

    def row_body(r, _):
      e = expr_v[j, r]
      for k in range(_D // 16):
        sl = pl.ds(k * 16, 16)
        rows_v[r, sl] = rows_v[r, sl] * e
      return 0

    lax.fori_loop(0, _CHUNK, row_body, 0, unroll=4)
    pltpu.sync_copy(rows_v, out_hbm.at[pl.ds(base + j * _CHUNK, _CHUNK)])

  pl.loop(0, nch)(chunk_body)


def kernel(gene_indices, expression_values, embedding_table):
  bsz, seq = gene_indices.shape
  tot = bsz * seq
  assert tot % (_NW * _CHUNK) == 0
  nch = tot // (_NW * _CHUNK)

  idx2d = gene_indices.reshape(_NW * nch, _CHUNK).astype(jnp.int32)
  expr2d = expression_values.reshape(_NW * nch, _CHUNK)

  mesh = plsc.VectorSubcoreMesh(core_axis_name="c", subcore_axis_name="s")
  out = pl.kernel(
      functools.partial(_body, nch),
      out_type=jax.ShapeDtypeStruct((tot, _D), jnp.float32),
      mesh=mesh,
      scratch_types=[
          pltpu.VMEM((nch, _CHUNK), jnp.int32),
          pltpu.VMEM((nch, _CHUNK), jnp.float32),
          pltpu.VMEM((_CHUNK, _D), jnp.float32),
          pltpu.SemaphoreType.DMA,
      ],
  )(idx2d, expr2d, embedding_table)
  return out.reshape(bsz, seq, _D)


# speedup vs baseline: 2.3889x; 2.3889x over previous
import functools

import jax
import jax.numpy as jnp
from jax import lax
from jax.experimental import pallas as pl
from jax.experimental.pallas import tpu as pltpu
from jax.experimental.pallas import tpu_sc as plsc

_D = 64
_NW = 32
_CHUNK = 128


def _body(nch, idx_hbm, expr_hbm, table_hbm, out_hbm, idx_v, expr_v, rows_v,
          sem):
  nc = plsc.get_sparse_core_info().num_cores
  wid = lax.axis_index("s") * nc + lax.axis_index("c")
  per_w = nch * _CHUNK
  base = wid * per_w

  pltpu.sync_copy(idx_hbm.at[pl.ds(wid * nch, nch)], idx_v)
  pltpu.sync_copy(expr_hbm.at[pl.ds(wid * nch, nch)], expr_v)

  def chunk_body(j):
    pltpu.async_copy(table_hbm.at[idx_v.at[j]], rows_v, sem).wait()

    def group_body(g, _):
      ev = expr_v[j, pl.ds(g * 16, 16)]
      for r in range(16):
        e = ev[r]
        row = g * 16 + r
        for k in range(_D // 16):
          sl = pl.ds(k * 16, 16)
          rows_v[row, sl] = rows_v[row, sl] * e
      return 0

    lax.fori_loop(0, _CHUNK // 16, group_body, 0)
    pltpu.sync_copy(rows_v, out_hbm.at[pl.ds(base + j * _CHUNK, _CHUNK)])

  pl.loop(0, nch)(chunk_body)


def kernel(gene_indices, expression_values, embedding_table):
  bsz, seq = gene_indices.shape
  tot = bsz * seq
  assert tot % (_NW * _CHUNK) == 0
  nch = tot // (_NW * _CHUNK)

  idx2d = gene_indices.reshape(_NW * nch, _CHUNK).astype(jnp.int32)
  expr2d = expression_values.reshape(_NW * nch, _CHUNK)

  mesh = plsc.VectorSubcoreMesh(core_axis_name="c", subcore_axis_name="s")
  out = pl.kernel(
      functools.partial(_body, nch),
      out_type=jax.ShapeDtypeStruct((tot, _D), jnp.float32),
      mesh=mesh,
      compiler_params=pltpu.CompilerParams(use_tc_tiling_on_sc=False),
      scratch_types=[
          pltpu.VMEM((nch, _CHUNK), jnp.int32),
          pltpu.VMEM((nch, _CHUNK), jnp.float32),
          pltpu.VMEM((_CHUNK, _D), jnp.float32),
          pltpu.SemaphoreType.DMA,
      ],
  )(idx2d, expr2d, embedding_table)
  return out.reshape(bsz, seq, _D)


# speedup vs baseline: 4.0733x; 1.7051x over previous
import functools

import jax
import jax.numpy as jnp
from jax import lax
from jax.experimental import pallas as pl
from jax.experimental.pallas import tpu as pltpu
from jax.experimental.pallas import tpu_sc as plsc

_D = 64
_NW = 32
_CHUNK = 128
_NBUF = 2


def _body(nch, idx_hbm, expr_hbm, table_hbm, out_hbm, idx_v, expr_v,
          g0, g1, s0, s1, gsem0, gsem1, ssem0, ssem1):
  nc = plsc.get_sparse_core_info().num_cores
  wid = lax.axis_index("s") * nc + lax.axis_index("c")
  per_w = nch * _CHUNK
  base = wid * per_w

  gbuf = (g0, g1)
  sbuf = (s0, s1)
  gsem = (gsem0, gsem1)
  ssem = (ssem0, ssem1)

  pltpu.sync_copy(idx_hbm.at[pl.ds(wid * nch, nch)], idx_v)
  pltpu.sync_copy(expr_hbm.at[pl.ds(wid * nch, nch)], expr_v)

  def gather(j, b):
    return pltpu.make_async_copy(table_hbm.at[idx_v.at[j]], gbuf[b], gsem[b])

  def scatter(j, b):
    return pltpu.make_async_copy(
        sbuf[b], out_hbm.at[pl.ds(base + j * _CHUNK, _CHUNK)], ssem[b])

  for b in range(_NBUF):
    gather(b, b).start()

  def outer(jo):
    for b in range(_NBUF):
      j = jo + b
      gather(j, b).wait()

      @pl.when(j >= _NBUF)
      def _():
        scatter(j - _NBUF, b).wait()

      def group_body(g, _):
        ev = expr_v[j, pl.ds(g * 16, 16)]
        for r in range(16):
          e = ev[r]
          row = g * 16 + r
          for k in range(_D // 16):
            sl = pl.ds(k * 16, 16)
            sbuf[b][row, sl] = gbuf[b][row, sl] * e
        return 0

      lax.fori_loop(0, _CHUNK // 16, group_body, 0)
      scatter(j, b).start()

      @pl.when(j + _NBUF < nch)
      def _():
        gather(j + _NBUF, b).start()

  pl.loop(0, nch, step=_NBUF)(outer)

  for b in range(_NBUF):
    scatter(nch - _NBUF + b, b).wait()


def kernel(gene_indices, expression_values, embedding_table):
  bsz, seq = gene_indices.shape
  tot = bsz * seq
  assert tot % (_NW * _CHUNK) == 0
  nch = tot // (_NW * _CHUNK)

  idx2d = gene_indices.reshape(_NW * nch, _CHUNK).astype(jnp.int32)
  expr2d = expression_values.reshape(_NW * nch, _CHUNK)

  mesh = plsc.VectorSubcoreMesh(core_axis_name="c", subcore_axis_name="s")
  out = pl.kernel(
      functools.partial(_body, nch),
      out_type=jax.ShapeDtypeStruct((tot, _D), jnp.float32),
      mesh=mesh,
      compiler_params=pltpu.CompilerParams(use_tc_tiling_on_sc=False),
      scratch_types=[
          pltpu.VMEM((nch, _CHUNK), jnp.int32),
          pltpu.VMEM((nch, _CHUNK), jnp.float32),
      ] + [pltpu.VMEM((_CHUNK, _D), jnp.float32)] * (2 * _NBUF)
        + [pltpu.SemaphoreType.DMA] * (2 * _NBUF),
  )(idx2d, expr2d, embedding_table)
  return out.reshape(bsz, seq, _D)


# speedup vs baseline: 4.1838x; 1.0271x over previous
import functools

import jax
import jax.numpy as jnp
from jax import lax
from jax.experimental import pallas as pl
from jax.experimental.pallas import tpu as pltpu
from jax.experimental.pallas import tpu_sc as plsc

_D = 64
_NW = 32
_NBUF = 2


def _body(nb, seq, idx_hbm, expr_hbm, table_hbm, out_hbm, idx_v, expr_v,
          g0, g1, s0, s1, gsem0, gsem1, ssem0, ssem1):
  nc = plsc.get_sparse_core_info().num_cores
  wid = lax.axis_index("s") * nc + lax.axis_index("c")
  base = wid * nb

  gbuf = (g0, g1)
  sbuf = (s0, s1)
  gsem = (gsem0, gsem1)
  ssem = (ssem0, ssem1)

  pltpu.sync_copy(idx_hbm.at[pl.ds(base, nb)], idx_v)
  pltpu.sync_copy(expr_hbm.at[pl.ds(base, nb)], expr_v)

  def gathers(i, b):
    return (
        pltpu.make_async_copy(table_hbm.at[idx_v.at[i, pl.ds(0, 128)]],
                              gbuf[b].at[pl.ds(0, 128)], gsem[b]),
        pltpu.make_async_copy(table_hbm.at[idx_v.at[i, pl.ds(128, seq - 128)]],
                              gbuf[b].at[pl.ds(128, seq - 128)], gsem[b]),
    )

  def scatter(i, b):
    return pltpu.make_async_copy(sbuf[b], out_hbm.at[base + i], ssem[b])

  def start_gathers(i, b):
    for c in gathers(i, b):
      c.start()

  def wait_gathers(i, b):
    for c in gathers(i, b):
      c.wait()

  for b in range(_NBUF):
    start_gathers(b, b)

  n_full = seq // 16
  tail = seq - n_full * 16
  tail_base = seq - 16

  def outer(io):
    for b in range(_NBUF):
      i = io + b
      wait_gathers(i, b)

      @pl.when(i >= _NBUF)
      def _():
        scatter(i - _NBUF, b).wait()

      def group_body(g, _):
        ev = expr_v[i, pl.ds(g * 16, 16)]
        for r in range(16):
          e = ev[r]
          row = g * 16 + r
          for k in range(_D // 16):
            sl = pl.ds(k * 16, 16)
            sbuf[b][row, sl] = gbuf[b][row, sl] * e
        return 0

      lax.fori_loop(0, n_full, group_body, 0)

      ev = expr_v[i, pl.ds(tail_base, 16)]
      for r in range(16 - tail, 16):
        e = ev[r]
        row = tail_base + r
        for k in range(_D // 16):
          sl = pl.ds(k * 16, 16)
          sbuf[b][row, sl] = gbuf[b][row, sl] * e

      scatter(i, b).start()

      @pl.when(i + _NBUF < nb)
      def _():
        start_gathers(i + _NBUF, b)

  pl.loop(0, nb, step=_NBUF)(outer)

  for b in range(_NBUF):
    scatter(nb - _NBUF + b, b).wait()


def kernel(gene_indices, expression_values, embedding_table):
  bsz, seq = gene_indices.shape
  assert bsz % _NW == 0 and seq == 200
  nb = bsz // _NW

  mesh = plsc.VectorSubcoreMesh(core_axis_name="c", subcore_axis_name="s")
  out = pl.kernel(
      functools.partial(_body, nb, seq),
      out_type=jax.ShapeDtypeStruct((bsz, seq, _D), jnp.float32),
      mesh=mesh,
      compiler_params=pltpu.CompilerParams(use_tc_tiling_on_sc=False),
      scratch_types=[
          pltpu.VMEM((nb, seq), jnp.int32),
          pltpu.VMEM((nb, seq), jnp.float32),
      ] + [pltpu.VMEM((seq, _D), jnp.float32)] * (2 * _NBUF)
        + [pltpu.SemaphoreType.DMA] * (2 * _NBUF),
  )(gene_indices.astype(jnp.int32), expression_values, embedding_table)
  return out
